# baseline (speedup 1.0000x reference)
import functools

import jax
import jax.numpy as jnp
from jax import lax
from jax.experimental import pallas as pl
from jax.experimental.pallas import tpu as pltpu
from jax.experimental.pallas import tpu_sc as plsc

NUM_CODES = 1024
DIM = 64
BETA = 0.25
ROWS = 2048
LANES = 16
HALF = DIM // 2


def _dist_block(z_ref, e_ref, iota_ref, idx_ref, loss_ref):
    i = pl.program_id(0)
    z = z_ref[...]
    e = e_ref[...]
    z_sq = jnp.sum(z * z, axis=1)
    e_sq = jnp.sum(e * e, axis=1)
    mm = jax.lax.dot_general(
        z, e, (((1,), (1,)), ((), ())),
        preferred_element_type=jnp.float32,
    )
    dist = (z_sq[:, None] + e_sq[None, :]) - 2.0 * mm
    minv = jnp.min(dist, axis=1)
    iota_f = iota_ref[0, 0]
    cand = jnp.where(dist == minv[:, None], iota_f[None, :],
                     jnp.float32(NUM_CODES - 1))
    idx_f = jnp.min(cand, axis=1)
    idx_ref[0, 0] = idx_f.astype(jnp.int32)
    part = jnp.sum(minv).reshape(1, 1)

    @pl.when(i == 0)
    def _init():
        loss_ref[...] = part

    @pl.when(i != 0)
    def _acc():
        loss_ref[...] += part


def _gather_body(e_hbm, idx_hbm, out_hbm, e_v, idx_v, out_v):
    wid = lax.axis_index("s") * 2 + lax.axis_index("c")
    b = wid // 2
    h = wid % 2
    pltpu.sync_copy(e_hbm, e_v)
    pltpu.sync_copy(idx_hbm.at[b], idx_v)
    base = jnp.int32(h * HALF)

    @plsc.parallel_loop(0, NUM_CODES // LANES, unroll=2)
    def step(j):
        col = pl.ds(j * LANES, LANES)
        flat0 = idx_v[col] * jnp.int32(DIM) + base
        for r in range(HALF):
            out_v[r, col] = plsc.load_gather(e_v, [flat0 + jnp.int32(r)])

    pltpu.sync_copy(out_v, out_hbm.at[b, pl.ds(h * HALF, HALF)])


@functools.partial(
    pl.kernel,
    mesh=plsc.VectorSubcoreMesh(core_axis_name="c", subcore_axis_name="s"),
    out_type=jax.ShapeDtypeStruct((16, DIM, NUM_CODES), jnp.float32),
    scratch_types=[
        pltpu.VMEM((NUM_CODES * DIM,), jnp.float32),
        pltpu.VMEM((NUM_CODES,), jnp.int32),
        pltpu.VMEM((HALF, NUM_CODES), jnp.float32),
    ],
    compiler_params=pltpu.CompilerParams(needs_layout_passes=False),
)
def _gather_kernel(e_hbm, idx_hbm, out_hbm, e_v, idx_v, out_v):
    _gather_body(e_hbm, idx_hbm, out_hbm, e_v, idx_v, out_v)


@jax.jit
def kernel(z_e, embedding):
    B, D = z_e.shape[0], z_e.shape[1]
    spatial = z_e.shape[2:]
    ndim = z_e.ndim
    perm = (0,) + tuple(range(2, ndim)) + (1,)
    z_flat = jnp.transpose(z_e, perm).reshape(-1, D)
    n = z_flat.shape[0]
    nb = n // ROWS
    iota_f = jnp.arange(NUM_CODES, dtype=jnp.float32).reshape(1, 1, NUM_CODES)
    idx, loss = pl.pallas_call(
        _dist_block,
        grid=(nb,),
        in_specs=[
            pl.BlockSpec((ROWS, D), lambda i: (i, 0)),
            pl.BlockSpec((NUM_CODES, D), lambda i: (0, 0)),
            pl.BlockSpec((1, 1, NUM_CODES), lambda i: (0, 0, 0)),
        ],
        out_specs=[
            pl.BlockSpec((1, 1, ROWS), lambda i: (i, 0, 0)),
            pl.BlockSpec((1, 1), lambda i: (0, 0)),
        ],
        out_shape=[
            jax.ShapeDtypeStruct((nb, 1, ROWS), jnp.int32),
            jax.ShapeDtypeStruct((1, 1), jnp.float32),
        ],
    )(z_flat, embedding, iota_f)
    idx2d = idx.reshape(B, NUM_CODES)
    zq = _gather_kernel(embedding.reshape(-1), idx2d)
    z_q_st = zq.reshape(z_e.shape)
    indices_map = idx2d.reshape((B,) + spatial)
    codebook_loss = loss[0, 0] / (n * D)
    return (
        z_q_st,
        indices_map,
        (1.0 + BETA) * codebook_loss,
        codebook_loss,
        BETA * codebook_loss,
    )


# speedup vs baseline: 1.5291x; 1.5291x over previous
import jax
import jax.numpy as jnp
from jax.experimental import pallas as pl

NUM_CODES = 1024
DIM = 64
BETA = 0.25
ROWS = 2048


def _vq_block(z_ref, e_ref, zq_ref, idx_ref, loss_ref):
    i = pl.program_id(0)
    z = z_ref[...]
    e = e_ref[...]
    z_sq = jnp.sum(z * z, axis=1)
    e_sq = jnp.sum(e * e, axis=1)
    mm = jax.lax.dot_general(
        z, e, (((1,), (1,)), ((), ())),
        preferred_element_type=jnp.float32,
    )
    dist = (z_sq[:, None] + e_sq[None, :]) - 2.0 * mm
    minv = jnp.min(dist, axis=1)
    iota = jax.lax.broadcasted_iota(jnp.int32, dist.shape, 1)
    onehot_raw = dist == minv[:, None]
    cand = jnp.where(onehot_raw, iota, jnp.int32(NUM_CODES))
    idx = jnp.min(cand, axis=1)
    onehot = (iota == idx[:, None]).astype(jnp.float32)
    z_q = jax.lax.dot_general(
        onehot, e, (((1,), (0,)), ((), ())),
        preferred_element_type=jnp.float32,
    )
    diff = z_q - z
    zq_ref[...] = z + diff
    idx_ref[0, 0] = idx
    part = jnp.sum(diff * diff).reshape(1, 1)

    @pl.when(i == 0)
    def _init():
        loss_ref[...] = part

    @pl.when(i != 0)
    def _acc():
        loss_ref[...] += part


@jax.jit
def kernel(z_e, embedding):
    B, D = z_e.shape[0], z_e.shape[1]
    spatial = z_e.shape[2:]
    ndim = z_e.ndim
    perm = (0,) + tuple(range(2, ndim)) + (1,)
    z_flat = jnp.transpose(z_e, perm).reshape(-1, D)
    n = z_flat.shape[0]
    nb = n // ROWS
    zq, idx, loss = pl.pallas_call(
        _vq_block,
        grid=(nb,),
        in_specs=[
            pl.BlockSpec((ROWS, D), lambda i: (i, 0)),
            pl.BlockSpec((NUM_CODES, D), lambda i: (0, 0)),
        ],
        out_specs=[
            pl.BlockSpec((ROWS, D), lambda i: (i, 0)),
            pl.BlockSpec((1, 1, ROWS), lambda i: (i, 0, 0)),
            pl.BlockSpec((1, 1), lambda i: (0, 0)),
        ],
        out_shape=[
            jax.ShapeDtypeStruct((n, D), jnp.float32),
            jax.ShapeDtypeStruct((nb, 1, ROWS), jnp.int32),
            jax.ShapeDtypeStruct((1, 1), jnp.float32),
        ],
    )(z_flat, embedding)
    inv_perm = (0, ndim - 1) + tuple(range(1, ndim - 1))
    z_q_st = jnp.transpose(zq.reshape((B,) + spatial + (D,)), inv_perm)
    indices_map = idx.reshape((B,) + spatial)
    codebook_loss = loss[0, 0] / (n * D)
    return (
        z_q_st,
        indices_map,
        (1.0 + BETA) * codebook_loss,
        codebook_loss,
        BETA * codebook_loss,
    )
